# baseline (speedup 1.0000x reference)
import functools

import numpy as np
import jax
import jax.numpy as jnp
from jax import lax
from jax.experimental import pallas as pl
from jax.experimental.pallas import tpu as pltpu

_REMAP_TO_80 = [1, 2, 3, 4, 5, 6, 7, 8, 9, 10, 11, 13, 14, 15, 16, 17, 18, 19,
                20, 21, 22, 23, 24, 25, 27, 28, 31, 32, 33, 34, 35, 36, 37, 38,
                39, 40, 41, 42, 43, 44, 46, 47, 48, 49, 50, 51, 52, 53, 54, 55,
                56, 57, 58, 59, 60, 61, 62, 63, 64, 65, 67, 70, 72, 73, 74, 75,
                76, 77, 78, 79, 80, 81, 82, 84, 85, 86, 87, 88, 89, 90]


def _resetter_body(bias_ref, selT_ref,
                   lg0_ref, lg1_ref, lg2_ref, lg3_ref, lg4_ref, lg5_ref,
                   bx_ref,
                   ol0_ref, ol1_ref, ol2_ref, ol3_ref, ol4_ref, ol5_ref,
                   obx_ref, oh_scr, *, k, nq, idx_bits, group):
    s = pl.program_id(1)
    cdims = (((1,), (1,)), ((), ()))
    selT = selT_ref[...]

    def gather_layer(lg_ref, ol_ref):
        for b in range(group):
            onehot = oh_scr[b * k:(b + 1) * k, :]
            g = lax.dot_general(lg_ref[:, b, :], onehot, cdims,
                                preferred_element_type=jnp.float32)
            ol_ref[b] = jnp.dot(selT, g,
                                preferred_element_type=jnp.float32)

    @pl.when(s == 0)
    def _():
        masked = lg0_ref[...] + bias_ref[...]
        scores = jnp.max(masked, axis=0)

        sbits = pltpu.bitcast(scores, jnp.int32)
        sortable = jnp.where(sbits < 0, sbits ^ jnp.int32(0x7FFFFFFF), sbits)
        qidx = lax.broadcasted_iota(jnp.int32, (group, nq), 1)
        keys = (((sortable >> idx_bits) << idx_bits)
                | (jnp.int32(nq - 1) - qidx))
        keysT = keys.T
        slot = lax.broadcasted_iota(jnp.int32, (k, 1), 0)

        for b in range(group):
            kc = keysT[:, b:b + 1]
            kr = keys[b:b + 1, :]
            rank = jnp.sum((kc > kr).astype(jnp.int32),
                           axis=0, keepdims=True)
            oh_scr[b * k:(b + 1) * k, :] = (rank == slot).astype(jnp.float32)
        gather_layer(lg0_ref, ol0_ref)

    for l, (lg_ref, ol_ref) in enumerate(
            ((lg1_ref, ol1_ref), (lg2_ref, ol2_ref), (lg3_ref, ol3_ref),
             (lg4_ref, ol4_ref), (lg5_ref, ol5_ref)), start=1):
        @pl.when(s == l)
        def _(lg_ref=lg_ref, ol_ref=ol_ref):
            gather_layer(lg_ref, ol_ref)

    @pl.when(s == 6)
    def _():
        for b in range(group):
            onehot = oh_scr[b * k:(b + 1) * k, :]
            obx_ref[b] = lax.dot_general(bx_ref[:, b, :], onehot, cdims,
                                         preferred_element_type=jnp.float32)


def kernel(pred_logits, pred_boxes, aux0_logits, aux0_boxes,
           aux1_logits, aux1_boxes, aux2_logits, aux2_boxes,
           aux3_logits, aux3_boxes, aux4_logits, aux4_boxes):
    logits_list = [pred_logits, aux0_logits, aux1_logits, aux2_logits,
                   aux3_logits, aux4_logits]
    boxes_list = [pred_boxes, aux0_boxes, aux1_boxes, aux2_boxes,
                  aux3_boxes, aux4_boxes]
    L = len(logits_list)
    bs, nq, C = pred_logits.shape
    k = 100
    idx_bits = max(1, int(nq - 1).bit_length())
    group = 8 if bs % 8 == 0 else 1
    n_groups = bs // group

    lgT_list = [jnp.transpose(lg.astype(jnp.float32), (2, 0, 1))
                for lg in logits_list]
    bxT = jnp.concatenate([jnp.transpose(bx.astype(jnp.float32), (2, 0, 1))
                           for bx in boxes_list], axis=0)

    selT_np = np.zeros((80, C), dtype=np.float32)
    selT_np[np.arange(80), np.asarray(_REMAP_TO_80)] = 1.0
    bias_np = np.full((C, 1, 1), -1e30, dtype=np.float32)
    bias_np[np.asarray(_REMAP_TO_80)] = 0.0
    selT = jnp.asarray(selT_np)
    bias = jnp.asarray(bias_np)

    body = functools.partial(_resetter_body, k=k, nq=nq, idx_bits=idx_bits,
                             group=group)

    def adv(g, cond):
        return jnp.where(cond, jnp.minimum(g + 1, n_groups - 1), g)

    def lg_spec(l):
        return pl.BlockSpec((C, group, nq),
                            lambda g, s, l=l: (0, adv(g, s > l), 0))

    def ol_spec(l):
        return pl.BlockSpec((group, 80, k),
                            lambda g, s, l=l: (adv(g, s > l), 0, 0))

    out = pl.pallas_call(
        body,
        out_shape=([jax.ShapeDtypeStruct((bs, 80, k), jnp.float32)
                    for _ in range(L)]
                   + [jax.ShapeDtypeStruct((bs, 4 * L, k), jnp.float32)]),
        grid=(n_groups, L + 1),
        in_specs=([pl.BlockSpec((C, 1, 1), lambda g, s: (0, 0, 0)),
                   pl.BlockSpec((80, C), lambda g, s: (0, 0))]
                  + [lg_spec(l) for l in range(L)]
                  + [pl.BlockSpec((4 * L, group, nq),
                                  lambda g, s: (0, g, 0))]),
        out_specs=([ol_spec(l) for l in range(L)]
                   + [pl.BlockSpec((group, 4 * L, k),
                                   lambda g, s: (g, 0, 0))]),
        scratch_shapes=[pltpu.VMEM((group * k, nq), jnp.float32)],
        compiler_params=pltpu.CompilerParams(
            dimension_semantics=("arbitrary", "arbitrary"),
            vmem_limit_bytes=50 * 1024 * 1024),
    )(bias, selT, *lgT_list, bxT)

    out_logits = [jnp.transpose(o, (0, 2, 1)) for o in out[:L]]
    bx_all = jnp.transpose(out[L], (0, 2, 1))
    result = {
        'pred_logits': out_logits[0].astype(pred_logits.dtype),
        'pred_boxes': bx_all[:, :, 0:4].astype(pred_boxes.dtype),
        'aux_outputs': [
            {'pred_logits': out_logits[1 + i].astype(pred_logits.dtype),
             'pred_boxes': bx_all[:, :, 4 * (1 + i):4 * (2 + i)]
                 .astype(pred_boxes.dtype)}
            for i in range(L - 1)
        ],
    }
    return result


# speedup vs baseline: 1.3350x; 1.3350x over previous
import functools

import numpy as np
import jax
import jax.numpy as jnp
from jax import lax
from jax.experimental import pallas as pl
from jax.experimental.pallas import tpu as pltpu

_REMAP_TO_80 = [1, 2, 3, 4, 5, 6, 7, 8, 9, 10, 11, 13, 14, 15, 16, 17, 18, 19,
                20, 21, 22, 23, 24, 25, 27, 28, 31, 32, 33, 34, 35, 36, 37, 38,
                39, 40, 41, 42, 43, 44, 46, 47, 48, 49, 50, 51, 52, 53, 54, 55,
                56, 57, 58, 59, 60, 61, 62, 63, 64, 65, 67, 70, 72, 73, 74, 75,
                76, 77, 78, 79, 80, 81, 82, 84, 85, 86, 87, 88, 89, 90]


def _resetter_body(bias_ref, selT_ref,
                   lg0_ref, lg1_ref, lg2_ref, lg3_ref, lg4_ref, lg5_ref,
                   bx_ref,
                   ol0_ref, ol1_ref, ol2_ref, ol3_ref, ol4_ref, ol5_ref,
                   obx_ref, *, k, nq, idx_bits, group):
    masked = lg0_ref[...] + bias_ref[...]
    scores = jnp.max(masked, axis=0)

    sbits = pltpu.bitcast(scores, jnp.int32)
    sortable = jnp.where(sbits < 0, sbits ^ jnp.int32(0x7FFFFFFF), sbits)
    qidx = lax.broadcasted_iota(jnp.int32, (group, nq), 1)
    keys = (((sortable >> idx_bits) << idx_bits)
            | (jnp.int32(nq - 1) - qidx))
    keysT = keys.T

    slot = lax.broadcasted_iota(jnp.int32, (k, 1), 0)
    selT = selT_ref[...]
    lg_refs = (lg0_ref, lg1_ref, lg2_ref, lg3_ref, lg4_ref, lg5_ref)
    ol_refs = (ol0_ref, ol1_ref, ol2_ref, ol3_ref, ol4_ref, ol5_ref)
    cdims = (((1,), (1,)), ((), ()))

    for b in range(group):
        kc = keysT[:, b:b + 1]
        kr = keys[b:b + 1, :]
        rank = jnp.sum((kc > kr).astype(jnp.int32),
                       axis=0, keepdims=True)
        onehot = (rank == slot).astype(jnp.float32)

        for lg_ref, ol_ref in zip(lg_refs, ol_refs):
            m1 = lax.dot_general(selT, lg_ref[:, b, :],
                                 (((1,), (0,)), ((), ())),
                                 preferred_element_type=jnp.float32)
            ol_ref[b] = lax.dot_general(m1, onehot, cdims,
                                        preferred_element_type=jnp.float32)
        obx_ref[b] = lax.dot_general(bx_ref[:, b, :], onehot, cdims,
                                     preferred_element_type=jnp.float32)


def kernel(pred_logits, pred_boxes, aux0_logits, aux0_boxes,
           aux1_logits, aux1_boxes, aux2_logits, aux2_boxes,
           aux3_logits, aux3_boxes, aux4_logits, aux4_boxes):
    logits_list = [pred_logits, aux0_logits, aux1_logits, aux2_logits,
                   aux3_logits, aux4_logits]
    boxes_list = [pred_boxes, aux0_boxes, aux1_boxes, aux2_boxes,
                  aux3_boxes, aux4_boxes]
    L = len(logits_list)
    bs, nq, C = pred_logits.shape
    k = 100
    idx_bits = max(1, int(nq - 1).bit_length())
    group = 8 if bs % 8 == 0 else 1
    n_groups = bs // group

    lgT_list = [jnp.transpose(lg.astype(jnp.float32), (2, 0, 1))
                for lg in logits_list]
    bxT = jnp.concatenate([jnp.transpose(bx.astype(jnp.float32), (2, 0, 1))
                           for bx in boxes_list], axis=0)

    selT_np = np.zeros((80, C), dtype=np.float32)
    selT_np[np.arange(80), np.asarray(_REMAP_TO_80)] = 1.0
    bias_np = np.full((C, 1, 1), -1e30, dtype=np.float32)
    bias_np[np.asarray(_REMAP_TO_80)] = 0.0
    selT = jnp.asarray(selT_np)
    bias = jnp.asarray(bias_np)

    body = functools.partial(_resetter_body, k=k, nq=nq, idx_bits=idx_bits,
                             group=group)

    in_spec = lambda c: pl.BlockSpec((c, group, nq), lambda gi: (0, gi, 0))
    out_spec = lambda c: pl.BlockSpec((group, c, k), lambda gi: (gi, 0, 0))
    out = pl.pallas_call(
        body,
        out_shape=([jax.ShapeDtypeStruct((bs, 80, k), jnp.float32)
                    for _ in range(L)]
                   + [jax.ShapeDtypeStruct((bs, 4 * L, k), jnp.float32)]),
        grid=(n_groups,),
        in_specs=([pl.BlockSpec((C, 1, 1), lambda gi: (0, 0, 0)),
                   pl.BlockSpec((80, C), lambda gi: (0, 0))]
                  + [in_spec(C) for _ in range(L)]
                  + [in_spec(4 * L)]),
        out_specs=([out_spec(80) for _ in range(L)] + [out_spec(4 * L)]),
        compiler_params=pltpu.CompilerParams(
            dimension_semantics=("arbitrary",),
            vmem_limit_bytes=50 * 1024 * 1024),
    )(bias, selT, *lgT_list, bxT)

    out_logits = [jnp.transpose(o, (0, 2, 1)) for o in out[:L]]
    bx_all = jnp.transpose(out[L], (0, 2, 1))
    result = {
        'pred_logits': out_logits[0].astype(pred_logits.dtype),
        'pred_boxes': bx_all[:, :, 0:4].astype(pred_boxes.dtype),
        'aux_outputs': [
            {'pred_logits': out_logits[1 + i].astype(pred_logits.dtype),
             'pred_boxes': bx_all[:, :, 4 * (1 + i):4 * (2 + i)]
                 .astype(pred_boxes.dtype)}
            for i in range(L - 1)
        ],
    }
    return result
